# baseline (speedup 1.0000x reference)
import functools

import jax
import jax.numpy as jnp
from jax import lax
from jax.experimental import pallas as pl
from jax.experimental.pallas import tpu as pltpu
from jax.experimental.pallas import tpu_sc as plsc

N_NODE = 10000
D = 128
E = 320000
NEG_SLOPE = 0.2

NW = 32
K = 64
NCHUNK = 162
EPW = NCHUNK * K
EPAD = NW * EPW
NPAD = 10240
RPT = NPAD // 16


def _k1_body(xa_ref, xl_ref, wsrc_ref, wdst_ref, attS_ref, attD_ref,
             h_ref, as_ref, ad_ref):
    h = jnp.dot(xa_ref[...], wsrc_ref[...], preferred_element_type=jnp.float32)
    h_ref[...] = h
    as_ref[...] = jnp.sum(h * attS_ref[...], axis=1, keepdims=True)
    hd = jnp.dot(xl_ref[...], wdst_ref[...], preferred_element_type=jnp.float32)
    ad_ref[...] = jnp.sum(hd * attD_ref[...], axis=1, keepdims=True)


def _dense_pre(x_attr, x_label, W_src, W_dst, att_src, att_dst):
    blk = 1000
    grid = N_NODE // blk
    return pl.pallas_call(
        _k1_body,
        grid=(grid,),
        in_specs=[
            pl.BlockSpec((blk, D), lambda i: (i, 0)),
            pl.BlockSpec((blk, D), lambda i: (i, 0)),
            pl.BlockSpec((D, D), lambda i: (0, 0)),
            pl.BlockSpec((D, D), lambda i: (0, 0)),
            pl.BlockSpec((1, D), lambda i: (0, 0)),
            pl.BlockSpec((1, D), lambda i: (0, 0)),
        ],
        out_specs=[
            pl.BlockSpec((blk, D), lambda i: (i, 0)),
            pl.BlockSpec((blk, 1), lambda i: (i, 0)),
            pl.BlockSpec((blk, 1), lambda i: (i, 0)),
        ],
        out_shape=[
            jax.ShapeDtypeStruct((N_NODE, D), jnp.float32),
            jax.ShapeDtypeStruct((N_NODE, 1), jnp.float32),
            jax.ShapeDtypeStruct((N_NODE, 1), jnp.float32),
        ],
    )(x_attr, x_label, W_src, W_dst,
      att_src.reshape(1, D), att_dst.reshape(1, D))


def _sc_body(h_hbm, srcs_hbm, dsts_hbm, asrc_hbm, adst_hbm,
             out_hbm, den_hbm,
             src_ch, dst_ch, asrc_v, adst_v, e_buf, rows, den_stage,
             out_acc, den_acc, sem_i,
             sem_g0, sem_g1, sem_g2, sem_s0, sem_s1, sem_s2,
             sem_e0, sem_e1, sem_e2):
    cid = lax.axis_index("c")
    sid = lax.axis_index("s")
    wid = sid * 2 + cid
    z16 = jnp.zeros((16,), jnp.float32)
    sems_g = [sem_g0, sem_g1, sem_g2]
    sems_s = [sem_s0, sem_s1, sem_s2]
    sems_e = [sem_e0, sem_e1, sem_e2]

    pltpu.async_copy(asrc_hbm, asrc_v, sem_g0)
    pltpu.async_copy(adst_hbm, adst_v, sem_g0)

    @plsc.parallel_loop(0, K)
    def _zrow(r):
        for j in range(8):
            rows[0, r, pl.ds(j * 16, 16)] = z16

    @plsc.parallel_loop(0, RPT // 16)
    def _zden(i):
        den_stage[pl.ds(i * 16, 16)] = z16

    for k in range(RPT // K):
        pltpu.sync_copy(rows.at[0], out_acc.at[pl.ds(sid * RPT + k * K, K)])
    pltpu.sync_copy(den_stage, den_acc.at[pl.ds(sid * RPT, RPT)])
    pltpu.make_async_copy(asrc_hbm, asrc_v, sem_g0).wait()
    pltpu.make_async_copy(adst_hbm, adst_v, sem_g0).wait()
    plsc.subcore_barrier()

    def _stage_idx(c, p):
        pltpu.async_copy(srcs_hbm.at[wid, c], src_ch.at[p], sem_i)
        pltpu.async_copy(dsts_hbm.at[wid, c], dst_ch.at[p], sem_i)

    def _wait_idx(c, p):
        pltpu.make_async_copy(srcs_hbm.at[wid, c], src_ch.at[p], sem_i).wait()
        pltpu.make_async_copy(dsts_hbm.at[wid, c], dst_ch.at[p], sem_i).wait()

    def _start_gather(p):
        pltpu.async_copy(h_hbm.at[src_ch.at[p]], rows.at[p], sems_g[p])

    def _wait_gather(p):
        pltpu.make_async_copy(
            h_hbm.at[src_ch.at[p]], rows.at[p], sems_g[p]).wait()

    def _wait_scatter(p):
        pltpu.make_async_copy(
            rows.at[p], out_acc.at[dst_ch.at[p]], sems_s[p]).wait()

    def _wait_escatter(p):
        pltpu.make_async_copy(
            e_buf.at[p], den_acc.at[dst_ch.at[p]], sems_e[p]).wait()

    _stage_idx(0, 0)
    _wait_idx(0, 0)
    _start_gather(0)

    def _chunk(c, p):
        q = (p + 1) % 3

        @pl.when(jnp.logical_and(c >= 2, c < NCHUNK - 1))
        def _():
            _wait_scatter(q)

        @pl.when(c < NCHUNK - 1)
        def _():
            _stage_idx(c + 1, q)

        @pl.when(c >= 3)
        def _():
            _wait_escatter(p)

        for j in range(K // 16):
            sv = src_ch[p, pl.ds(j * 16, 16)]
            dv = dst_ch[p, pl.ds(j * 16, 16)]
            a_s = plsc.load_gather(asrc_v, [sv])
            a_d = plsc.load_gather(adst_v, [dv])
            t = a_s + a_d
            alpha = jnp.where(t > 0, t, NEG_SLOPE * t)
            ev = jnp.exp(alpha)
            gid = (wid * EPW + c * K + j * 16) + lax.iota(jnp.int32, 16)
            ev = jnp.where(gid < E, ev, 0.0)
            e_buf[p, pl.ds(j * 16, 16)] = ev

        pltpu.async_copy(e_buf.at[p], den_acc.at[dst_ch.at[p]], sems_e[p],
                         add=True)
        _wait_gather(p)

        @plsc.parallel_loop(0, K, unroll=4)
        def _row(r):
            eb = plsc.load_gather(e_buf.at[p], [jnp.full((16,), r, jnp.int32)])
            for f in range(8):
                rows[p, r, pl.ds(f * 16, 16)] = (
                    rows[p, r, pl.ds(f * 16, 16)] * eb)

        @pl.when(c < NCHUNK - 1)
        def _():
            _wait_idx(c + 1, q)
            _start_gather(q)

        pltpu.async_copy(rows.at[p], out_acc.at[dst_ch.at[p]], sems_s[p],
                         add=True)

    def _triple(i, carry):
        c0 = i * 3
        _chunk(c0, 0)
        _chunk(c0 + 1, 1)
        _chunk(c0 + 2, 2)
        return carry
    lax.fori_loop(0, NCHUNK // 3, _triple, 0)

    for p in range(3):
        _wait_scatter(p)
        _wait_escatter(p)

    plsc.subcore_barrier()
    for k in range(RPT // K):
        r0 = sid * RPT + k * K
        pltpu.sync_copy(out_acc.at[pl.ds(r0, K)], rows.at[0])
        pltpu.sync_copy(rows.at[0], out_hbm.at[cid, pl.ds(r0, K)])
    pltpu.sync_copy(den_acc.at[pl.ds(sid * RPT, RPT)], den_stage)
    pltpu.sync_copy(den_stage, den_hbm.at[cid, pl.ds(sid * RPT, RPT)])


def _sc_aggregate(h_src, srcs, dsts, a_src, a_dst):
    mesh = plsc.VectorSubcoreMesh(core_axis_name="c", subcore_axis_name="s")
    fn = pl.kernel(
        _sc_body,
        out_type=[
            jax.ShapeDtypeStruct((2, NPAD, D), jnp.float32),
            jax.ShapeDtypeStruct((2, NPAD), jnp.float32),
        ],
        mesh=mesh,
        compiler_params=pltpu.CompilerParams(needs_layout_passes=False),
        scratch_types=[
            pltpu.VMEM((3, K), jnp.int32),
            pltpu.VMEM((3, K), jnp.int32),
            pltpu.VMEM((N_NODE,), jnp.float32),
            pltpu.VMEM((N_NODE,), jnp.float32),
            pltpu.VMEM((3, K), jnp.float32),
            pltpu.VMEM((3, K, D), jnp.float32),
            pltpu.VMEM((RPT,), jnp.float32),
            pltpu.VMEM_SHARED((NPAD, D), jnp.float32),
            pltpu.VMEM_SHARED((NPAD,), jnp.float32),
        ] + [pltpu.SemaphoreType.DMA] * 10,
    )
    return fn(h_src, srcs, dsts, a_src, a_dst)


def _k3_body(p0_ref, p1_ref, d0_ref, d1_ref, bias_ref, wlin_ref, blin_ref,
             out_ref):
    d = d0_ref[...] + d1_ref[...] + 1e-16
    h = jnp.maximum((p0_ref[...] + p1_ref[...]) / d + bias_ref[...], 0.0)
    out_ref[...] = (
        jnp.dot(h, wlin_ref[...], preferred_element_type=jnp.float32)
        + blin_ref[...])


def _dense_post(p0, p1, d0, d1, bias, W_lin, b_lin):
    blk = 1000
    grid = N_NODE // blk
    return pl.pallas_call(
        _k3_body,
        grid=(grid,),
        in_specs=[
            pl.BlockSpec((blk, D), lambda i: (i, 0)),
            pl.BlockSpec((blk, D), lambda i: (i, 0)),
            pl.BlockSpec((blk, 1), lambda i: (i, 0)),
            pl.BlockSpec((blk, 1), lambda i: (i, 0)),
            pl.BlockSpec((1, D), lambda i: (0, 0)),
            pl.BlockSpec((D, D), lambda i: (0, 0)),
            pl.BlockSpec((1, D), lambda i: (0, 0)),
        ],
        out_specs=pl.BlockSpec((blk, D), lambda i: (i, 0)),
        out_shape=jax.ShapeDtypeStruct((N_NODE, D), jnp.float32),
    )(p0, p1, d0, d1, bias, W_lin, b_lin)


def kernel(x_label, x_attr, edge_index_l2a, edge_index_a2l,
           W_src_l2a, W_dst_l2a, att_src_l2a, att_dst_l2a, bias_l2a,
           W_src_a2l, W_dst_a2l, att_src_a2l, att_dst_a2l, bias_a2l,
           W_lin, b_lin):
    h_src, a_src, a_dst = _dense_pre(
        x_attr, x_label, W_src_a2l, W_dst_a2l, att_src_a2l, att_dst_a2l)

    src = edge_index_a2l[0]
    dst = edge_index_a2l[1]
    pad = (jnp.arange(EPAD - E, dtype=jnp.int32) * 37) % N_NODE
    srcs = jnp.concatenate([src, pad]).reshape(NW, NCHUNK, K)
    dsts = jnp.concatenate([dst, pad]).reshape(NW, NCHUNK, K)

    out_part, den_part = _sc_aggregate(
        h_src, srcs, dsts, a_src.reshape(-1), a_dst.reshape(-1))

    return _dense_post(
        out_part[0], out_part[1],
        den_part[0].reshape(NPAD, 1), den_part[1].reshape(NPAD, 1),
        bias_a2l.reshape(1, D), W_lin, b_lin.reshape(1, D))


# speedup vs baseline: 1.4566x; 1.4566x over previous
import functools

import jax
import jax.numpy as jnp
from jax import lax
from jax.experimental import pallas as pl
from jax.experimental.pallas import tpu as pltpu
from jax.experimental.pallas import tpu_sc as plsc

N_NODE = 10000
D = 128
E = 320000
NEG_SLOPE = 0.2

NW = 32
K = 64
NCHUNK = 162
EPW = NCHUNK * K
EPAD = NW * EPW
NPAD = 10240
RPT = NPAD // 16


def _k1_body(xa_ref, xl_ref, wsrc_ref, wdst_ref, attS_ref, attD_ref,
             h_ref, as_ref, ad_ref):
    h = jnp.dot(xa_ref[...], wsrc_ref[...], preferred_element_type=jnp.float32)
    h_ref[...] = h
    as_ref[...] = jnp.sum(h * attS_ref[...], axis=1, keepdims=True)
    hd = jnp.dot(xl_ref[...], wdst_ref[...], preferred_element_type=jnp.float32)
    ad_ref[...] = jnp.sum(hd * attD_ref[...], axis=1, keepdims=True)


def _dense_pre(x_attr, x_label, W_src, W_dst, att_src, att_dst):
    blk = 1000
    grid = N_NODE // blk
    return pl.pallas_call(
        _k1_body,
        grid=(grid,),
        in_specs=[
            pl.BlockSpec((blk, D), lambda i: (i, 0)),
            pl.BlockSpec((blk, D), lambda i: (i, 0)),
            pl.BlockSpec((D, D), lambda i: (0, 0)),
            pl.BlockSpec((D, D), lambda i: (0, 0)),
            pl.BlockSpec((1, D), lambda i: (0, 0)),
            pl.BlockSpec((1, D), lambda i: (0, 0)),
        ],
        out_specs=[
            pl.BlockSpec((blk, D), lambda i: (i, 0)),
            pl.BlockSpec((blk, 1), lambda i: (i, 0)),
            pl.BlockSpec((blk, 1), lambda i: (i, 0)),
        ],
        out_shape=[
            jax.ShapeDtypeStruct((N_NODE, D), jnp.float32),
            jax.ShapeDtypeStruct((N_NODE, 1), jnp.float32),
            jax.ShapeDtypeStruct((N_NODE, 1), jnp.float32),
        ],
    )(x_attr, x_label, W_src, W_dst,
      att_src.reshape(1, D), att_dst.reshape(1, D))


def _sc_body(h_hbm, srcs_hbm, dsts_hbm, asrc_hbm, adst_hbm,
             out_hbm, den_hbm,
             src_ch, dst_ch, dst_sc, asrc_v, adst_v, e_buf, rows, den_stage,
             out_acc, den_acc, sem_i,
             sem_g0, sem_g1, sem_g2, sem_s0, sem_s1, sem_s2,
             sem_e0, sem_e1, sem_e2):
    cid = lax.axis_index("c")
    sid = lax.axis_index("s")
    wid = sid * 2 + cid
    z16 = jnp.zeros((16,), jnp.float32)
    sems_g = [sem_g0, sem_g1, sem_g2]
    sems_s = [sem_s0, sem_s1, sem_s2]
    sems_e = [sem_e0, sem_e1, sem_e2]

    pltpu.async_copy(asrc_hbm, asrc_v, sem_g0)
    pltpu.async_copy(adst_hbm, adst_v, sem_g0)

    @plsc.parallel_loop(0, K)
    def _zrow(r):
        for j in range(8):
            rows[0, r, pl.ds(j * 16, 16)] = z16

    @plsc.parallel_loop(0, RPT // 16)
    def _zden(i):
        den_stage[pl.ds(i * 16, 16)] = z16

    for k in range(RPT // K):
        pltpu.sync_copy(rows.at[0], out_acc.at[pl.ds(sid * RPT + k * K, K)])
    pltpu.sync_copy(den_stage, den_acc.at[pl.ds(sid * RPT, RPT)])
    pltpu.make_async_copy(asrc_hbm, asrc_v, sem_g0).wait()
    pltpu.make_async_copy(adst_hbm, adst_v, sem_g0).wait()
    plsc.subcore_barrier()

    def _stage_idx(c, p):
        pltpu.async_copy(srcs_hbm.at[wid, c], src_ch.at[p], sem_i)
        pltpu.async_copy(dsts_hbm.at[wid, c], dst_ch.at[p], sem_i)

    def _wait_idx(c, p):
        pltpu.make_async_copy(srcs_hbm.at[wid, c], src_ch.at[p], sem_i).wait()
        pltpu.make_async_copy(dsts_hbm.at[wid, c], dst_ch.at[p], sem_i).wait()

    def _start_gather(p):
        pltpu.async_copy(h_hbm.at[src_ch.at[p]], rows.at[p], sems_g[p])

    def _wait_gather(p):
        pltpu.make_async_copy(
            h_hbm.at[src_ch.at[p]], rows.at[p], sems_g[p]).wait()

    def _wait_scatter(p):
        pltpu.make_async_copy(
            rows.at[p], out_acc.at[dst_sc.at[p]], sems_s[p]).wait()

    def _wait_escatter(p):
        pltpu.make_async_copy(
            e_buf.at[p], den_acc.at[dst_sc.at[p]], sems_e[p]).wait()

    _stage_idx(0, 0)
    _wait_idx(0, 0)
    _start_gather(0)
    _stage_idx(1, 1)
    _wait_idx(1, 1)
    _start_gather(1)

    def _chunk(c, p):
        r2 = (p + 2) % 3

        @pl.when(c < NCHUNK - 2)
        def _():
            _stage_idx(c + 2, r2)

        @pl.when(c >= 3)
        def _():
            _wait_escatter(p)

        for j in range(K // 16):
            sv = src_ch[p, pl.ds(j * 16, 16)]
            dv = dst_ch[p, pl.ds(j * 16, 16)]
            a_s = plsc.load_gather(asrc_v, [sv])
            a_d = plsc.load_gather(adst_v, [dv])
            t = a_s + a_d
            alpha = jnp.where(t > 0, t, NEG_SLOPE * t)
            ev = jnp.exp(alpha)
            gid = (wid * EPW + c * K + j * 16) + lax.iota(jnp.int32, 16)
            ev = jnp.where(gid < E, ev, 0.0)
            e_buf[p, pl.ds(j * 16, 16)] = ev
            dst_sc[p, pl.ds(j * 16, 16)] = dv

        pltpu.async_copy(e_buf.at[p], den_acc.at[dst_sc.at[p]], sems_e[p],
                         add=True)
        _wait_gather(p)

        @plsc.parallel_loop(0, K, unroll=4)
        def _row(r):
            eb = plsc.load_gather(e_buf.at[p], [jnp.full((16,), r, jnp.int32)])
            for f in range(8):
                rows[p, r, pl.ds(f * 16, 16)] = (
                    rows[p, r, pl.ds(f * 16, 16)] * eb)

        @pl.when(c >= 1)
        def _():
            _wait_scatter(r2)

        @pl.when(c < NCHUNK - 2)
        def _():
            _wait_idx(c + 2, r2)
            _start_gather(r2)

        pltpu.async_copy(rows.at[p], out_acc.at[dst_sc.at[p]], sems_s[p],
                         add=True)

    def _triple(i, carry):
        c0 = i * 3
        _chunk(c0, 0)
        _chunk(c0 + 1, 1)
        _chunk(c0 + 2, 2)
        return carry
    lax.fori_loop(0, NCHUNK // 3, _triple, 0)

    _wait_scatter((NCHUNK - 1) % 3)
    for p in range(3):
        _wait_escatter(p)

    plsc.subcore_barrier()
    for k in range(RPT // K):
        r0 = sid * RPT + k * K
        pltpu.sync_copy(out_acc.at[pl.ds(r0, K)], rows.at[0])
        pltpu.sync_copy(rows.at[0], out_hbm.at[cid, pl.ds(r0, K)])
    pltpu.sync_copy(den_acc.at[pl.ds(sid * RPT, RPT)], den_stage)
    pltpu.sync_copy(den_stage, den_hbm.at[cid, pl.ds(sid * RPT, RPT)])


def _sc_aggregate(h_src, srcs, dsts, a_src, a_dst):
    mesh = plsc.VectorSubcoreMesh(core_axis_name="c", subcore_axis_name="s")
    fn = pl.kernel(
        _sc_body,
        out_type=[
            jax.ShapeDtypeStruct((2, NPAD, D), jnp.float32),
            jax.ShapeDtypeStruct((2, NPAD), jnp.float32),
        ],
        mesh=mesh,
        compiler_params=pltpu.CompilerParams(needs_layout_passes=False),
        scratch_types=[
            pltpu.VMEM((3, K), jnp.int32),
            pltpu.VMEM((3, K), jnp.int32),
            pltpu.VMEM((3, K), jnp.int32),
            pltpu.VMEM((N_NODE,), jnp.float32),
            pltpu.VMEM((N_NODE,), jnp.float32),
            pltpu.VMEM((3, K), jnp.float32),
            pltpu.VMEM((3, K, D), jnp.float32),
            pltpu.VMEM((RPT,), jnp.float32),
            pltpu.VMEM_SHARED((NPAD, D), jnp.float32),
            pltpu.VMEM_SHARED((NPAD,), jnp.float32),
        ] + [pltpu.SemaphoreType.DMA] * 10,
    )
    return fn(h_src, srcs, dsts, a_src, a_dst)


def _k3_body(p0_ref, p1_ref, d0_ref, d1_ref, bias_ref, wlin_ref, blin_ref,
             out_ref):
    d = d0_ref[...] + d1_ref[...] + 1e-16
    h = jnp.maximum((p0_ref[...] + p1_ref[...]) / d + bias_ref[...], 0.0)
    out_ref[...] = (
        jnp.dot(h, wlin_ref[...], preferred_element_type=jnp.float32)
        + blin_ref[...])


def _dense_post(p0, p1, d0, d1, bias, W_lin, b_lin):
    blk = 1000
    grid = N_NODE // blk
    return pl.pallas_call(
        _k3_body,
        grid=(grid,),
        in_specs=[
            pl.BlockSpec((blk, D), lambda i: (i, 0)),
            pl.BlockSpec((blk, D), lambda i: (i, 0)),
            pl.BlockSpec((blk, 1), lambda i: (i, 0)),
            pl.BlockSpec((blk, 1), lambda i: (i, 0)),
            pl.BlockSpec((1, D), lambda i: (0, 0)),
            pl.BlockSpec((D, D), lambda i: (0, 0)),
            pl.BlockSpec((1, D), lambda i: (0, 0)),
        ],
        out_specs=pl.BlockSpec((blk, D), lambda i: (i, 0)),
        out_shape=jax.ShapeDtypeStruct((N_NODE, D), jnp.float32),
    )(p0, p1, d0, d1, bias, W_lin, b_lin)


def kernel(x_label, x_attr, edge_index_l2a, edge_index_a2l,
           W_src_l2a, W_dst_l2a, att_src_l2a, att_dst_l2a, bias_l2a,
           W_src_a2l, W_dst_a2l, att_src_a2l, att_dst_a2l, bias_a2l,
           W_lin, b_lin):
    h_src, a_src, a_dst = _dense_pre(
        x_attr, x_label, W_src_a2l, W_dst_a2l, att_src_a2l, att_dst_a2l)

    src = edge_index_a2l[0]
    dst = edge_index_a2l[1]
    pad = (jnp.arange(EPAD - E, dtype=jnp.int32) * 37) % N_NODE
    srcs = jnp.concatenate([src, pad]).reshape(NW, NCHUNK, K)
    dsts = jnp.concatenate([dst, pad]).reshape(NW, NCHUNK, K)

    out_part, den_part = _sc_aggregate(
        h_src, srcs, dsts, a_src.reshape(-1), a_dst.reshape(-1))

    return _dense_post(
        out_part[0], out_part[1],
        den_part[0].reshape(NPAD, 1), den_part[1].reshape(NPAD, 1),
        bias_a2l.reshape(1, D), W_lin, b_lin.reshape(1, D))


# speedup vs baseline: 1.4818x; 1.0173x over previous
import functools

import jax
import jax.numpy as jnp
from jax import lax
from jax.experimental import pallas as pl
from jax.experimental.pallas import tpu as pltpu
from jax.experimental.pallas import tpu_sc as plsc

N_NODE = 10000
D = 128
E = 320000
NEG_SLOPE = 0.2

NW = 32
K = 64
NCHUNK = 162
EPW = NCHUNK * K
EPAD = NW * EPW
NPAD = 10240
RPT = NPAD // 16


def _k1_body(xa_ref, xl_ref, wsrc_ref, wdst_ref, attS_ref, attD_ref,
             h_ref, as_ref, ad_ref):
    h = jnp.dot(xa_ref[...], wsrc_ref[...], preferred_element_type=jnp.float32)
    h_ref[...] = h
    as_ref[...] = jnp.sum(h * attS_ref[...], axis=1, keepdims=True)
    hd = jnp.dot(xl_ref[...], wdst_ref[...], preferred_element_type=jnp.float32)
    ad_ref[...] = jnp.sum(hd * attD_ref[...], axis=1, keepdims=True)


def _dense_pre(x_attr, x_label, W_src, W_dst, att_src, att_dst):
    blk = 1000
    grid = N_NODE // blk
    return pl.pallas_call(
        _k1_body,
        grid=(grid,),
        in_specs=[
            pl.BlockSpec((blk, D), lambda i: (i, 0)),
            pl.BlockSpec((blk, D), lambda i: (i, 0)),
            pl.BlockSpec((D, D), lambda i: (0, 0)),
            pl.BlockSpec((D, D), lambda i: (0, 0)),
            pl.BlockSpec((1, D), lambda i: (0, 0)),
            pl.BlockSpec((1, D), lambda i: (0, 0)),
        ],
        out_specs=[
            pl.BlockSpec((blk, D), lambda i: (i, 0)),
            pl.BlockSpec((blk, 1), lambda i: (i, 0)),
            pl.BlockSpec((blk, 1), lambda i: (i, 0)),
        ],
        out_shape=[
            jax.ShapeDtypeStruct((N_NODE, D), jnp.float32),
            jax.ShapeDtypeStruct((N_NODE, 1), jnp.float32),
            jax.ShapeDtypeStruct((N_NODE, 1), jnp.float32),
        ],
    )(x_attr, x_label, W_src, W_dst,
      att_src.reshape(1, D), att_dst.reshape(1, D))


def _sc_body(h_hbm, srcs_hbm, dsts_hbm, asrc_hbm, adst_hbm,
             out_hbm, den_hbm,
             src_ch, dst_ch, dst_sc, asrc_v, adst_v, e_buf, rows, den_stage,
             out_acc, den_acc, sem_i,
             sem_g0, sem_g1, sem_g2, sem_s0, sem_s1, sem_s2,
             sem_e0, sem_e1, sem_e2):
    cid = lax.axis_index("c")
    sid = lax.axis_index("s")
    wid = sid * 2 + cid
    z16 = jnp.zeros((16,), jnp.float32)
    sems_g = [sem_g0, sem_g1, sem_g2]
    sems_s = [sem_s0, sem_s1, sem_s2]
    sems_e = [sem_e0, sem_e1, sem_e2]

    pltpu.async_copy(asrc_hbm, asrc_v, sem_g0)
    pltpu.async_copy(adst_hbm, adst_v, sem_g0)

    @plsc.parallel_loop(0, K)
    def _zrow(r):
        for j in range(8):
            rows[0, r, pl.ds(j * 16, 16)] = z16

    @plsc.parallel_loop(0, RPT // 16)
    def _zden(i):
        den_stage[pl.ds(i * 16, 16)] = z16

    for k in range(RPT // K):
        pltpu.sync_copy(rows.at[0], out_acc.at[pl.ds(sid * RPT + k * K, K)])
    pltpu.sync_copy(den_stage, den_acc.at[pl.ds(sid * RPT, RPT)])
    pltpu.make_async_copy(asrc_hbm, asrc_v, sem_g0).wait()
    pltpu.make_async_copy(adst_hbm, adst_v, sem_g0).wait()
    plsc.subcore_barrier()

    def _stage_idx(c, p):
        pltpu.async_copy(srcs_hbm.at[wid, c], src_ch.at[p], sem_i)
        pltpu.async_copy(dsts_hbm.at[wid, c], dst_ch.at[p], sem_i)

    def _wait_idx(c, p):
        pltpu.make_async_copy(srcs_hbm.at[wid, c], src_ch.at[p], sem_i).wait()
        pltpu.make_async_copy(dsts_hbm.at[wid, c], dst_ch.at[p], sem_i).wait()

    def _start_gather(p):
        pltpu.async_copy(h_hbm.at[src_ch.at[p]], rows.at[p], sems_g[p])

    def _wait_gather(p):
        pltpu.make_async_copy(
            h_hbm.at[src_ch.at[p]], rows.at[p], sems_g[p]).wait()

    def _wait_scatter(p):
        pltpu.make_async_copy(
            rows.at[p], out_acc.at[dst_sc.at[p]], sems_s[p]).wait()

    def _wait_escatter(p):
        pltpu.make_async_copy(
            e_buf.at[p], den_acc.at[dst_sc.at[p]], sems_e[p]).wait()

    _stage_idx(0, 0)
    _wait_idx(0, 0)
    _start_gather(0)
    _stage_idx(1, 1)
    _wait_idx(1, 1)
    _start_gather(1)

    def _chunk(c, p):
        r2 = (p + 2) % 3

        @pl.when(c < NCHUNK - 2)
        def _():
            _stage_idx(c + 2, r2)

        @pl.when(c >= 3)
        def _():
            _wait_escatter(p)

        for j in range(K // 16):
            sv = src_ch[p, pl.ds(j * 16, 16)]
            dv = dst_ch[p, pl.ds(j * 16, 16)]
            a_s = plsc.load_gather(asrc_v, [sv])
            a_d = plsc.load_gather(adst_v, [dv])
            t = a_s + a_d
            alpha = jnp.where(t > 0, t, NEG_SLOPE * t)
            ev = jnp.exp(alpha)
            gid = (wid * EPW + c * K + j * 16) + lax.iota(jnp.int32, 16)
            ev = jnp.where(gid < E, ev, 0.0)
            e_buf[p, pl.ds(j * 16, 16)] = ev
            dst_sc[p, pl.ds(j * 16, 16)] = dv

        pltpu.async_copy(e_buf.at[p], den_acc.at[dst_sc.at[p]], sems_e[p],
                         add=True)
        _wait_gather(p)

        @plsc.parallel_loop(0, K, unroll=4)
        def _row(r):
            eb = plsc.load_gather(e_buf.at[p], [jnp.full((16,), r, jnp.int32)])
            for f in range(8):
                rows[p, r, pl.ds(f * 16, 16)] = (
                    rows[p, r, pl.ds(f * 16, 16)] * eb)

        @pl.when(c >= 1)
        def _():
            _wait_scatter(r2)

        @pl.when(c < NCHUNK - 2)
        def _():
            _wait_idx(c + 2, r2)
            _start_gather(r2)

        pltpu.async_copy(rows.at[p], out_acc.at[dst_sc.at[p]], sems_s[p],
                         add=True)

    def _triple(i, carry):
        c0 = i * 3
        _chunk(c0, 0)
        _chunk(c0 + 1, 1)
        _chunk(c0 + 2, 2)
        return carry
    lax.fori_loop(0, NCHUNK // 3, _triple, 0)

    _wait_scatter((NCHUNK - 1) % 3)
    for p in range(3):
        _wait_escatter(p)

    plsc.subcore_barrier()
    for k in range(RPT // K):
        r0 = sid * RPT + k * K
        pltpu.sync_copy(out_acc.at[pl.ds(r0, K)], rows.at[0])
        pltpu.sync_copy(rows.at[0], out_hbm.at[cid, pl.ds(r0, K)])
    pltpu.sync_copy(den_acc.at[pl.ds(sid * RPT, RPT)], den_stage)
    pltpu.sync_copy(den_stage, den_hbm.at[cid, pl.ds(sid * RPT, RPT)])


def _sc_aggregate(h_src, srcs, dsts, a_src, a_dst):
    mesh = plsc.VectorSubcoreMesh(core_axis_name="c", subcore_axis_name="s")
    fn = pl.kernel(
        _sc_body,
        out_type=[
            jax.ShapeDtypeStruct((2, NPAD, D), jnp.float32),
            jax.ShapeDtypeStruct((2, NPAD), jnp.float32),
        ],
        mesh=mesh,
        compiler_params=pltpu.CompilerParams(needs_layout_passes=False),
        scratch_types=[
            pltpu.VMEM((3, K), jnp.int32),
            pltpu.VMEM((3, K), jnp.int32),
            pltpu.VMEM((3, K), jnp.int32),
            pltpu.VMEM((N_NODE,), jnp.float32),
            pltpu.VMEM((N_NODE,), jnp.float32),
            pltpu.VMEM((3, K), jnp.float32),
            pltpu.VMEM((3, K, D), jnp.float32),
            pltpu.VMEM((RPT,), jnp.float32),
            pltpu.VMEM_SHARED((NPAD, D), jnp.float32),
            pltpu.VMEM_SHARED((NPAD,), jnp.float32),
        ] + [pltpu.SemaphoreType.DMA] * 10,
    )
    return fn(h_src, srcs, dsts, a_src, a_dst)


def _k3_body(p_ref, d_ref, bias_ref, wlin_ref, blin_ref, out_ref):
    d = d_ref[0] + d_ref[1] + 1e-16
    h = jnp.maximum((p_ref[0] + p_ref[1]) / d + bias_ref[...], 0.0)
    out_ref[...] = (
        jnp.dot(h, wlin_ref[...], preferred_element_type=jnp.float32)
        + blin_ref[...])


def _dense_post(parts, dens, bias, W_lin, b_lin):
    blk = 1000
    grid = N_NODE // blk
    return pl.pallas_call(
        _k3_body,
        grid=(grid,),
        in_specs=[
            pl.BlockSpec((2, blk, D), lambda i: (0, i, 0)),
            pl.BlockSpec((2, blk, 1), lambda i: (0, i, 0)),
            pl.BlockSpec((1, D), lambda i: (0, 0)),
            pl.BlockSpec((D, D), lambda i: (0, 0)),
            pl.BlockSpec((1, D), lambda i: (0, 0)),
        ],
        out_specs=pl.BlockSpec((blk, D), lambda i: (i, 0)),
        out_shape=jax.ShapeDtypeStruct((N_NODE, D), jnp.float32),
    )(parts, dens, bias, W_lin, b_lin)


def kernel(x_label, x_attr, edge_index_l2a, edge_index_a2l,
           W_src_l2a, W_dst_l2a, att_src_l2a, att_dst_l2a, bias_l2a,
           W_src_a2l, W_dst_a2l, att_src_a2l, att_dst_a2l, bias_a2l,
           W_lin, b_lin):
    h_src, a_src, a_dst = _dense_pre(
        x_attr, x_label, W_src_a2l, W_dst_a2l, att_src_a2l, att_dst_a2l)

    src = edge_index_a2l[0]
    dst = edge_index_a2l[1]
    pad = (jnp.arange(EPAD - E, dtype=jnp.int32) * 37) % N_NODE
    srcs = jnp.concatenate([src, pad]).reshape(NW, NCHUNK, K)
    dsts = jnp.concatenate([dst, pad]).reshape(NW, NCHUNK, K)

    out_part, den_part = _sc_aggregate(
        h_src, srcs, dsts, a_src.reshape(-1), a_dst.reshape(-1))

    return _dense_post(
        out_part, den_part.reshape(2, NPAD, 1),
        bias_a2l.reshape(1, D), W_lin, b_lin.reshape(1, D))
